# Initial kernel scaffold; baseline (speedup 1.0000x reference)
import jax
import jax.numpy as jnp
from jax.experimental import pallas as pl


def kernel(x, edge_index, W1, b1, W2, b2):
    raise NotImplementedError("write your pallas kernel here")


# speedup vs baseline: 13.3954x; 13.3954x over previous
import functools

import jax
import jax.numpy as jnp
from jax import lax
from jax.experimental import pallas as pl
from jax.experimental.pallas import tpu as pltpu
from jax.experimental.pallas import tpu_sc as plsc

N = 10000
E = 320000
D = 128
H = 128
C = 40
C_PAD = 64

NC = 2
NS = 16
NW = NC * NS
EK = 80
E_PER_W = E // NW
N_CHUNKS = E_PER_W // EK
R_BIG = 632
R_LAST = N - (NS - 1) * R_BIG
DEG_PAD = 10240
DEG_PER_TILE = DEG_PAD // NS

_MESH = dict(core_axis_name="c", subcore_axis_name="s", num_cores=NC,
             num_subcores=NS)


def _sc_degree(dst):
    @functools.partial(
        pl.kernel,
        out_type=jax.ShapeDtypeStruct((NC * DEG_PAD,), jnp.float32),
        mesh=plsc.VectorSubcoreMesh(**_MESH),
        scratch_types=[
            pltpu.VMEM((EK,), jnp.int32),
            pltpu.VMEM((EK,), jnp.float32),
            pltpu.VMEM((DEG_PER_TILE,), jnp.float32),
            pltpu.VMEM_SHARED((DEG_PAD,), jnp.float32),
        ],
    )
    def deg_kernel(dst_hbm, out_hbm, dst_v, ones_v, zbuf, deg_sh):
        cid = lax.axis_index("c")
        sid = lax.axis_index("s")
        w = cid * NS + sid

        for j in range(EK // 16):
            ones_v[pl.ds(j * 16, 16)] = jnp.ones((16,), jnp.float32)
        for j in range(DEG_PER_TILE // 16):
            zbuf[pl.ds(j * 16, 16)] = jnp.zeros((16,), jnp.float32)
        pltpu.sync_copy(zbuf, deg_sh.at[pl.ds(sid * DEG_PER_TILE,
                                              DEG_PER_TILE)])
        plsc.subcore_barrier()

        def body(i, carry):
            base = w * E_PER_W + i * EK
            pltpu.sync_copy(dst_hbm.at[pl.ds(base, EK)], dst_v)
            pltpu.sync_copy(ones_v, deg_sh.at[dst_v], add=True)
            return carry

        lax.fori_loop(0, N_CHUNKS, body, 0)
        plsc.subcore_barrier()
        pltpu.sync_copy(
            deg_sh.at[pl.ds(sid * DEG_PER_TILE, DEG_PER_TILE)],
            out_hbm.at[pl.ds(cid * DEG_PAD + sid * DEG_PER_TILE,
                             DEG_PER_TILE)])

    return deg_kernel(dst).reshape(NC, DEG_PAD)


def _sc_aggregate(y, src, dst, f):
    @functools.partial(
        pl.kernel,
        out_type=jax.ShapeDtypeStruct((NC * N, f), jnp.float32),
        mesh=plsc.VectorSubcoreMesh(**_MESH),
        compiler_params=pltpu.CompilerParams(use_tc_tiling_on_sc=False),
        scratch_types=[
            pltpu.VMEM((EK,), jnp.int32),
            pltpu.VMEM((EK,), jnp.int32),
            pltpu.VMEM((EK, f), jnp.float32),
            pltpu.VMEM_SHARED((N, f), jnp.float32),
        ],
    )
    def agg_kernel(y_hbm, src_hbm, dst_hbm, out_hbm, src_v, dst_v, rows_v,
                   agg_sh):
        cid = lax.axis_index("c")
        sid = lax.axis_index("s")
        w = cid * NS + sid

        r0 = sid * R_BIG

        @pl.when(sid < NS - 1)
        def _():
            pltpu.sync_copy(y_hbm.at[pl.ds(r0, R_BIG)],
                            agg_sh.at[pl.ds(r0, R_BIG)])

        @pl.when(sid == NS - 1)
        def _():
            pltpu.sync_copy(y_hbm.at[pl.ds((NS - 1) * R_BIG, R_LAST)],
                            agg_sh.at[pl.ds((NS - 1) * R_BIG, R_LAST)])

        plsc.subcore_barrier()

        def body(i, carry):
            base = w * E_PER_W + i * EK
            pltpu.sync_copy(src_hbm.at[pl.ds(base, EK)], src_v)
            pltpu.sync_copy(dst_hbm.at[pl.ds(base, EK)], dst_v)
            pltpu.sync_copy(y_hbm.at[src_v], rows_v)
            pltpu.sync_copy(rows_v, agg_sh.at[dst_v], add=True)
            return carry

        lax.fori_loop(0, N_CHUNKS, body, 0)
        plsc.subcore_barrier()

        @pl.when(sid < NS - 1)
        def _():
            pltpu.sync_copy(agg_sh.at[pl.ds(r0, R_BIG)],
                            out_hbm.at[pl.ds(cid * N + r0, R_BIG)])

        @pl.when(sid == NS - 1)
        def _():
            pltpu.sync_copy(
                agg_sh.at[pl.ds((NS - 1) * R_BIG, R_LAST)],
                out_hbm.at[pl.ds(cid * N + (NS - 1) * R_BIG, R_LAST)])

    return agg_kernel(y, src, dst).reshape(NC, N, f)


def _tc_dinv(deg2):
    def body(deg_ref, out_ref):
        s = deg_ref[0:1, :] + deg_ref[1:2, :] + 1.0
        out_ref[...] = lax.rsqrt(s)

    return pl.pallas_call(
        body, out_shape=jax.ShapeDtypeStruct((1, DEG_PAD), jnp.float32),
    )(deg2)


_BLK = 400
_GRID = N // _BLK


def _tc_scale_matmul(x, w, dinv_col):
    def body(x_ref, w_ref, d_ref, y_ref):
        xw = jnp.dot(x_ref[...], w_ref[...],
                     preferred_element_type=jnp.float32)
        y_ref[...] = xw * d_ref[...]

    f_in, f_out = w.shape
    return pl.pallas_call(
        body,
        grid=(_GRID,),
        in_specs=[
            pl.BlockSpec((_BLK, f_in), lambda i: (i, 0)),
            pl.BlockSpec((f_in, f_out), lambda i: (0, 0)),
            pl.BlockSpec((_BLK, 1), lambda i: (i, 0)),
        ],
        out_specs=pl.BlockSpec((_BLK, f_out), lambda i: (i, 0)),
        out_shape=jax.ShapeDtypeStruct((N, f_out), jnp.float32),
    )(x, w, dinv_col)


def _tc_combine_relu_matmul(agg2, y, dinv_col, b_row, w2):
    def body(a0_ref, a1_ref, y_ref, d_ref, b_ref, w_ref, out_ref):
        t = (a0_ref[...] + a1_ref[...] - y_ref[...]) * d_ref[...] + b_ref[...]
        h = jnp.maximum(t, 0.0)
        out_ref[...] = jnp.dot(h, w_ref[...],
                               preferred_element_type=jnp.float32) * d_ref[...]

    f_in, f_out = w2.shape
    flat = agg2.reshape(NC * N, f_in)
    return pl.pallas_call(
        body,
        grid=(_GRID,),
        in_specs=[
            pl.BlockSpec((_BLK, f_in), lambda i: (i, 0)),
            pl.BlockSpec((_BLK, f_in), lambda i: (i + _GRID, 0)),
            pl.BlockSpec((_BLK, f_in), lambda i: (i, 0)),
            pl.BlockSpec((_BLK, 1), lambda i: (i, 0)),
            pl.BlockSpec((1, f_in), lambda i: (0, 0)),
            pl.BlockSpec((f_in, f_out), lambda i: (0, 0)),
        ],
        out_specs=pl.BlockSpec((_BLK, f_out), lambda i: (i, 0)),
        out_shape=jax.ShapeDtypeStruct((N, f_out), jnp.float32),
    )(flat, flat, y, dinv_col, b_row, w2)


def _tc_combine_final(agg2, y, dinv_col, b_row):
    def body(a0_ref, a1_ref, y_ref, d_ref, b_ref, out_ref):
        out_ref[...] = ((a0_ref[...] + a1_ref[...] - y_ref[...]) * d_ref[...]
                        + b_ref[...])

    f = y.shape[1]
    flat = agg2.reshape(NC * N, f)
    return pl.pallas_call(
        body,
        grid=(_GRID,),
        in_specs=[
            pl.BlockSpec((_BLK, f), lambda i: (i, 0)),
            pl.BlockSpec((_BLK, f), lambda i: (i + _GRID, 0)),
            pl.BlockSpec((_BLK, f), lambda i: (i, 0)),
            pl.BlockSpec((_BLK, 1), lambda i: (i, 0)),
            pl.BlockSpec((1, f), lambda i: (0, 0)),
        ],
        out_specs=pl.BlockSpec((_BLK, f), lambda i: (i, 0)),
        out_shape=jax.ShapeDtypeStruct((N, f), jnp.float32),
    )(flat, flat, y, dinv_col, b_row)


def kernel(x, edge_index, W1, b1, W2, b2):
    src = edge_index[0]
    dst = edge_index[1]

    deg2 = _sc_degree(dst)
    dinv_row = _tc_dinv(deg2)
    dinv_col = dinv_row.reshape(DEG_PAD, 1)[:N]

    y1 = _tc_scale_matmul(x, W1, dinv_col)
    agg1 = _sc_aggregate(y1, src, dst, H)

    W2p = jnp.zeros((H, C_PAD), jnp.float32).at[:, :C].set(W2)
    b2p = jnp.zeros((1, C_PAD), jnp.float32).at[0, :C].set(b2)
    y2 = _tc_combine_relu_matmul(agg1, y1, dinv_col, b1.reshape(1, H), W2p)
    agg2 = _sc_aggregate(y2, src, dst, C_PAD)

    out = _tc_combine_final(agg2, y2, dinv_col, b2p)
    return out[:, :C]


# speedup vs baseline: 29.8721x; 2.2300x over previous
import functools

import jax
import jax.numpy as jnp
from jax import lax
from jax.experimental import pallas as pl
from jax.experimental.pallas import tpu as pltpu
from jax.experimental.pallas import tpu_sc as plsc

N = 10000
E = 320000
D = 128
H = 128
C = 40
C_PAD = 64

NC = 2
NS = 16
NW = NC * NS
EK = 80
E_PER_W = E // NW
N_CHUNKS = E_PER_W // EK
R_BIG = 632
R_LAST = N - (NS - 1) * R_BIG
DEG_PAD = 10240
DEG_PER_TILE = DEG_PAD // NS

_MESH = dict(core_axis_name="c", subcore_axis_name="s", num_cores=NC,
             num_subcores=NS)


def _sc_degree(dst3):
    @functools.partial(
        pl.kernel,
        out_type=jax.ShapeDtypeStruct((NC * DEG_PAD,), jnp.float32),
        mesh=plsc.VectorSubcoreMesh(**_MESH),
        scratch_types=[
            pltpu.VMEM((N_CHUNKS, EK), jnp.int32),
            pltpu.VMEM((EK,), jnp.float32),
            pltpu.VMEM((DEG_PER_TILE,), jnp.float32),
            pltpu.VMEM_SHARED((DEG_PAD,), jnp.float32),
        ],
    )
    def deg_kernel(dst_hbm, out_hbm, dst_all, ones_v, zbuf, deg_sh):
        cid = lax.axis_index("c")
        sid = lax.axis_index("s")
        w = cid * NS + sid

        for j in range(EK // 16):
            ones_v[pl.ds(j * 16, 16)] = jnp.ones((16,), jnp.float32)
        for j in range(DEG_PER_TILE // 16):
            zbuf[pl.ds(j * 16, 16)] = jnp.zeros((16,), jnp.float32)
        pltpu.sync_copy(zbuf, deg_sh.at[pl.ds(sid * DEG_PER_TILE,
                                              DEG_PER_TILE)])
        pltpu.sync_copy(dst_hbm.at[w], dst_all)
        plsc.subcore_barrier()

        def body(i, carry):
            pltpu.sync_copy(ones_v, deg_sh.at[dst_all.at[i]], add=True)
            return carry

        lax.fori_loop(0, N_CHUNKS, body, 0)
        plsc.subcore_barrier()
        pltpu.sync_copy(
            deg_sh.at[pl.ds(sid * DEG_PER_TILE, DEG_PER_TILE)],
            out_hbm.at[pl.ds(cid * DEG_PAD + sid * DEG_PER_TILE,
                             DEG_PER_TILE)])

    return deg_kernel(dst3).reshape(NC, DEG_PAD)


def _sc_aggregate(y, src3, dst3, f):
    @functools.partial(
        pl.kernel,
        out_type=jax.ShapeDtypeStruct((NC * N, f), jnp.float32),
        mesh=plsc.VectorSubcoreMesh(**_MESH),
        compiler_params=pltpu.CompilerParams(use_tc_tiling_on_sc=False),
        scratch_types=[
            pltpu.VMEM((N_CHUNKS, EK), jnp.int32),
            pltpu.VMEM((N_CHUNKS, EK), jnp.int32),
            pltpu.VMEM((EK, f), jnp.float32),
            pltpu.VMEM((EK, f), jnp.float32),
            pltpu.VMEM_SHARED((N, f), jnp.float32),
            pltpu.SemaphoreType.DMA,
            pltpu.SemaphoreType.DMA,
        ],
    )
    def agg_kernel(y_hbm, src_hbm, dst_hbm, out_hbm, src_all, dst_all,
                   rows0, rows1, agg_sh, sem0, sem1):
        cid = lax.axis_index("c")
        sid = lax.axis_index("s")
        w = cid * NS + sid

        r0 = sid * R_BIG

        @pl.when(sid < NS - 1)
        def _():
            pltpu.sync_copy(y_hbm.at[pl.ds(r0, R_BIG)],
                            agg_sh.at[pl.ds(r0, R_BIG)])

        @pl.when(sid == NS - 1)
        def _():
            pltpu.sync_copy(y_hbm.at[pl.ds((NS - 1) * R_BIG, R_LAST)],
                            agg_sh.at[pl.ds((NS - 1) * R_BIG, R_LAST)])

        pltpu.sync_copy(src_hbm.at[w], src_all)
        pltpu.sync_copy(dst_hbm.at[w], dst_all)
        plsc.subcore_barrier()

        pltpu.async_copy(y_hbm.at[src_all.at[0]], rows0, sem0)

        def body(j, carry):
            i0 = 2 * j
            pltpu.async_copy(y_hbm.at[src_all.at[i0 + 1]], rows1, sem1)
            pltpu.make_async_copy(y_hbm.at[src_all.at[i0]], rows0, sem0).wait()
            pltpu.sync_copy(rows0, agg_sh.at[dst_all.at[i0]], add=True)
            pltpu.async_copy(y_hbm.at[src_all.at[i0 + 2]], rows0, sem0)
            pltpu.make_async_copy(y_hbm.at[src_all.at[i0 + 1]], rows1,
                                  sem1).wait()
            pltpu.sync_copy(rows1, agg_sh.at[dst_all.at[i0 + 1]], add=True)
            return carry

        lax.fori_loop(0, (N_CHUNKS - 1) // 2, body, 0)
        pltpu.make_async_copy(y_hbm.at[src_all.at[N_CHUNKS - 1]], rows0,
                              sem0).wait()
        pltpu.sync_copy(rows0, agg_sh.at[dst_all.at[N_CHUNKS - 1]], add=True)
        plsc.subcore_barrier()

        @pl.when(sid < NS - 1)
        def _():
            pltpu.sync_copy(agg_sh.at[pl.ds(r0, R_BIG)],
                            out_hbm.at[pl.ds(cid * N + r0, R_BIG)])

        @pl.when(sid == NS - 1)
        def _():
            pltpu.sync_copy(
                agg_sh.at[pl.ds((NS - 1) * R_BIG, R_LAST)],
                out_hbm.at[pl.ds(cid * N + (NS - 1) * R_BIG, R_LAST)])

    return agg_kernel(y, src3, dst3).reshape(NC, N, f)


def _tc_dinv(deg2):
    def body(deg_ref, out_ref):
        s = deg_ref[0:1, :] + deg_ref[1:2, :] + 1.0
        out_ref[...] = lax.rsqrt(s)

    return pl.pallas_call(
        body, out_shape=jax.ShapeDtypeStruct((1, DEG_PAD), jnp.float32),
    )(deg2)


_BLK = 400
_GRID = N // _BLK


def _tc_scale_matmul(x, w, dinv_col):
    def body(x_ref, w_ref, d_ref, y_ref):
        xw = jnp.dot(x_ref[...], w_ref[...],
                     preferred_element_type=jnp.float32)
        y_ref[...] = xw * d_ref[...]

    f_in, f_out = w.shape
    return pl.pallas_call(
        body,
        grid=(_GRID,),
        in_specs=[
            pl.BlockSpec((_BLK, f_in), lambda i: (i, 0)),
            pl.BlockSpec((f_in, f_out), lambda i: (0, 0)),
            pl.BlockSpec((_BLK, 1), lambda i: (i, 0)),
        ],
        out_specs=pl.BlockSpec((_BLK, f_out), lambda i: (i, 0)),
        out_shape=jax.ShapeDtypeStruct((N, f_out), jnp.float32),
    )(x, w, dinv_col)


def _tc_combine_relu_matmul(agg2, y, dinv_col, b_row, w2):
    def body(a0_ref, a1_ref, y_ref, d_ref, b_ref, w_ref, out_ref):
        t = (a0_ref[...] + a1_ref[...] - y_ref[...]) * d_ref[...] + b_ref[...]
        h = jnp.maximum(t, 0.0)
        out_ref[...] = jnp.dot(h, w_ref[...],
                               preferred_element_type=jnp.float32) * d_ref[...]

    f_in, f_out = w2.shape
    flat = agg2.reshape(NC * N, f_in)
    return pl.pallas_call(
        body,
        grid=(_GRID,),
        in_specs=[
            pl.BlockSpec((_BLK, f_in), lambda i: (i, 0)),
            pl.BlockSpec((_BLK, f_in), lambda i: (i + _GRID, 0)),
            pl.BlockSpec((_BLK, f_in), lambda i: (i, 0)),
            pl.BlockSpec((_BLK, 1), lambda i: (i, 0)),
            pl.BlockSpec((1, f_in), lambda i: (0, 0)),
            pl.BlockSpec((f_in, f_out), lambda i: (0, 0)),
        ],
        out_specs=pl.BlockSpec((_BLK, f_out), lambda i: (i, 0)),
        out_shape=jax.ShapeDtypeStruct((N, f_out), jnp.float32),
    )(flat, flat, y, dinv_col, b_row, w2)


def _tc_combine_final(agg2, y, dinv_col, b_row):
    def body(a0_ref, a1_ref, y_ref, d_ref, b_ref, out_ref):
        out_ref[...] = ((a0_ref[...] + a1_ref[...] - y_ref[...]) * d_ref[...]
                        + b_ref[...])

    f = y.shape[1]
    flat = agg2.reshape(NC * N, f)
    return pl.pallas_call(
        body,
        grid=(_GRID,),
        in_specs=[
            pl.BlockSpec((_BLK, f), lambda i: (i, 0)),
            pl.BlockSpec((_BLK, f), lambda i: (i + _GRID, 0)),
            pl.BlockSpec((_BLK, f), lambda i: (i, 0)),
            pl.BlockSpec((_BLK, 1), lambda i: (i, 0)),
            pl.BlockSpec((1, f), lambda i: (0, 0)),
        ],
        out_specs=pl.BlockSpec((_BLK, f), lambda i: (i, 0)),
        out_shape=jax.ShapeDtypeStruct((N, f), jnp.float32),
    )(flat, flat, y, dinv_col, b_row)


def kernel(x, edge_index, W1, b1, W2, b2):
    src3 = edge_index[0].reshape(NW, N_CHUNKS, EK)
    dst3 = edge_index[1].reshape(NW, N_CHUNKS, EK)

    deg2 = _sc_degree(dst3)
    dinv_row = _tc_dinv(deg2)
    dinv_col = dinv_row.reshape(DEG_PAD, 1)[:N]

    y1 = _tc_scale_matmul(x, W1, dinv_col)
    agg1 = _sc_aggregate(y1, src3, dst3, H)

    W2p = jnp.zeros((H, C_PAD), jnp.float32).at[:, :C].set(W2)
    b2p = jnp.zeros((1, C_PAD), jnp.float32).at[0, :C].set(b2)
    y2 = _tc_combine_relu_matmul(agg1, y1, dinv_col, b1.reshape(1, H), W2p)
    agg2 = _sc_aggregate(y2, src3, dst3, C_PAD)

    out = _tc_combine_final(agg2, y2, dinv_col, b2p)
    return out[:, :C]


# speedup vs baseline: 30.0200x; 1.0050x over previous
import functools

import jax
import jax.numpy as jnp
from jax import lax
from jax.experimental import pallas as pl
from jax.experimental.pallas import tpu as pltpu
from jax.experimental.pallas import tpu_sc as plsc

N = 10000
E = 320000
D = 128
H = 128
C = 40
C_PAD = 64

NC = 2
NS = 16
NW = NC * NS
EK = 80
E_PER_W = E // NW
N_CHUNKS = E_PER_W // EK
R_BIG = 632
R_LAST = N - (NS - 1) * R_BIG
DEG_PAD = 10240
DEG_PER_TILE = DEG_PAD // NS

_MESH = dict(core_axis_name="c", subcore_axis_name="s", num_cores=NC,
             num_subcores=NS)


def _sc_degree(dst3):
    @functools.partial(
        pl.kernel,
        out_type=jax.ShapeDtypeStruct((NC * DEG_PAD,), jnp.float32),
        mesh=plsc.VectorSubcoreMesh(**_MESH),
        scratch_types=[
            pltpu.VMEM((N_CHUNKS, EK), jnp.int32),
            pltpu.VMEM((EK,), jnp.float32),
            pltpu.VMEM((DEG_PER_TILE,), jnp.float32),
            pltpu.VMEM_SHARED((DEG_PAD,), jnp.float32),
        ],
    )
    def deg_kernel(dst_hbm, out_hbm, dst_all, ones_v, zbuf, deg_sh):
        cid = lax.axis_index("c")
        sid = lax.axis_index("s")
        w = cid * NS + sid

        for j in range(EK // 16):
            ones_v[pl.ds(j * 16, 16)] = jnp.ones((16,), jnp.float32)
        for j in range(DEG_PER_TILE // 16):
            zbuf[pl.ds(j * 16, 16)] = jnp.zeros((16,), jnp.float32)
        pltpu.sync_copy(zbuf, deg_sh.at[pl.ds(sid * DEG_PER_TILE,
                                              DEG_PER_TILE)])
        pltpu.sync_copy(dst_hbm.at[w], dst_all)
        plsc.subcore_barrier()

        def body(i, carry):
            pltpu.sync_copy(ones_v, deg_sh.at[dst_all.at[i]], add=True)
            return carry

        lax.fori_loop(0, N_CHUNKS, body, 0)
        plsc.subcore_barrier()
        pltpu.sync_copy(
            deg_sh.at[pl.ds(sid * DEG_PER_TILE, DEG_PER_TILE)],
            out_hbm.at[pl.ds(cid * DEG_PAD + sid * DEG_PER_TILE,
                             DEG_PER_TILE)])

    return deg_kernel(dst3).reshape(NC, DEG_PAD)


def _sc_aggregate(y, src3, dst3, f):
    @functools.partial(
        pl.kernel,
        out_type=jax.ShapeDtypeStruct((NC * N, f), jnp.float32),
        mesh=plsc.VectorSubcoreMesh(**_MESH),
        compiler_params=pltpu.CompilerParams(use_tc_tiling_on_sc=False),
        scratch_types=[
            pltpu.VMEM((N_CHUNKS, EK), jnp.int32),
            pltpu.VMEM((N_CHUNKS, EK), jnp.int32),
            pltpu.VMEM((EK, f), jnp.float32),
            pltpu.VMEM((EK, f), jnp.float32),
            pltpu.VMEM_SHARED((N, f), jnp.float32),
            pltpu.SemaphoreType.DMA,
            pltpu.SemaphoreType.DMA,
        ],
    )
    def agg_kernel(y_hbm, src_hbm, dst_hbm, out_hbm, src_all, dst_all,
                   rows0, rows1, agg_sh, sem0, sem1):
        cid = lax.axis_index("c")
        sid = lax.axis_index("s")
        w = cid * NS + sid

        r0 = sid * R_BIG

        @pl.when(sid < NS - 1)
        def _():
            pltpu.sync_copy(y_hbm.at[pl.ds(r0, R_BIG)],
                            agg_sh.at[pl.ds(r0, R_BIG)])

        @pl.when(sid == NS - 1)
        def _():
            pltpu.sync_copy(y_hbm.at[pl.ds((NS - 1) * R_BIG, R_LAST)],
                            agg_sh.at[pl.ds((NS - 1) * R_BIG, R_LAST)])

        pltpu.sync_copy(src_hbm.at[w], src_all)
        pltpu.sync_copy(dst_hbm.at[w], dst_all)
        plsc.subcore_barrier()

        pltpu.async_copy(y_hbm.at[src_all.at[0]], rows0, sem0)

        def body(j, carry):
            i0 = 2 * j
            pltpu.async_copy(y_hbm.at[src_all.at[i0 + 1]], rows1, sem1)
            pltpu.make_async_copy(y_hbm.at[src_all.at[i0]], rows0, sem0).wait()
            pltpu.sync_copy(rows0, agg_sh.at[dst_all.at[i0]], add=True)
            pltpu.async_copy(y_hbm.at[src_all.at[i0 + 2]], rows0, sem0)
            pltpu.make_async_copy(y_hbm.at[src_all.at[i0 + 1]], rows1,
                                  sem1).wait()
            pltpu.sync_copy(rows1, agg_sh.at[dst_all.at[i0 + 1]], add=True)
            return carry

        lax.fori_loop(0, (N_CHUNKS - 1) // 2, body, 0)
        pltpu.make_async_copy(y_hbm.at[src_all.at[N_CHUNKS - 1]], rows0,
                              sem0).wait()
        pltpu.sync_copy(rows0, agg_sh.at[dst_all.at[N_CHUNKS - 1]], add=True)
        plsc.subcore_barrier()

        @pl.when(sid < NS - 1)
        def _():
            pltpu.sync_copy(agg_sh.at[pl.ds(r0, R_BIG)],
                            out_hbm.at[pl.ds(cid * N + r0, R_BIG)])

        @pl.when(sid == NS - 1)
        def _():
            pltpu.sync_copy(
                agg_sh.at[pl.ds((NS - 1) * R_BIG, R_LAST)],
                out_hbm.at[pl.ds(cid * N + (NS - 1) * R_BIG, R_LAST)])

    return agg_kernel(y, src3, dst3).reshape(NC, N, f)


_BLK = 400
_GRID = N // _BLK


def _dinv_block(d_ref):
    return lax.rsqrt(d_ref[:, 0:1] + d_ref[:, 1:2] + 1.0)


def _tc_scale(x, deg_col):
    def body(d_ref, x_ref, u_ref):
        u_ref[...] = x_ref[...] * _dinv_block(d_ref)

    return pl.pallas_call(
        body,
        grid=(_GRID,),
        in_specs=[
            pl.BlockSpec((_BLK, 2), lambda i: (i, 0)),
            pl.BlockSpec((_BLK, D), lambda i: (i, 0)),
        ],
        out_specs=pl.BlockSpec((_BLK, D), lambda i: (i, 0)),
        out_shape=jax.ShapeDtypeStruct((N, D), jnp.float32),
    )(deg_col, x)


def _tc_mid(agg2, u, deg_col, w1, b1_row, w2p):
    def body(a0_ref, a1_ref, u_ref, d_ref, w1_ref, b1_ref, w2_ref, out_ref):
        dinv = _dinv_block(d_ref)
        v = a0_ref[...] + a1_ref[...] - u_ref[...]
        t = jnp.dot(v, w1_ref[...],
                    preferred_element_type=jnp.float32) * dinv + b1_ref[...]
        h = jnp.maximum(t, 0.0)
        out_ref[...] = jnp.dot(h, w2_ref[...],
                               preferred_element_type=jnp.float32) * dinv

    f_out = w2p.shape[1]
    flat = agg2.reshape(NC * N, D)
    return pl.pallas_call(
        body,
        grid=(_GRID,),
        in_specs=[
            pl.BlockSpec((_BLK, D), lambda i: (i, 0)),
            pl.BlockSpec((_BLK, D), lambda i: (i + _GRID, 0)),
            pl.BlockSpec((_BLK, D), lambda i: (i, 0)),
            pl.BlockSpec((_BLK, 2), lambda i: (i, 0)),
            pl.BlockSpec((D, D), lambda i: (0, 0)),
            pl.BlockSpec((1, D), lambda i: (0, 0)),
            pl.BlockSpec((D, f_out), lambda i: (0, 0)),
        ],
        out_specs=pl.BlockSpec((_BLK, f_out), lambda i: (i, 0)),
        out_shape=jax.ShapeDtypeStruct((N, f_out), jnp.float32),
    )(flat, flat, u, deg_col, w1, b1_row, w2p)


def _tc_combine_final(agg2, y, deg_col, b_row):
    def body(a0_ref, a1_ref, y_ref, d_ref, b_ref, out_ref):
        out_ref[...] = ((a0_ref[...] + a1_ref[...] - y_ref[...])
                        * _dinv_block(d_ref) + b_ref[...])

    f = y.shape[1]
    flat = agg2.reshape(NC * N, f)
    return pl.pallas_call(
        body,
        grid=(_GRID,),
        in_specs=[
            pl.BlockSpec((_BLK, f), lambda i: (i, 0)),
            pl.BlockSpec((_BLK, f), lambda i: (i + _GRID, 0)),
            pl.BlockSpec((_BLK, f), lambda i: (i, 0)),
            pl.BlockSpec((_BLK, 2), lambda i: (i, 0)),
            pl.BlockSpec((1, f), lambda i: (0, 0)),
        ],
        out_specs=pl.BlockSpec((_BLK, f), lambda i: (i, 0)),
        out_shape=jax.ShapeDtypeStruct((N, f), jnp.float32),
    )(flat, flat, y, deg_col, b_row)


def kernel(x, edge_index, W1, b1, W2, b2):
    src3 = edge_index[0].reshape(NW, N_CHUNKS, EK)
    dst3 = edge_index[1].reshape(NW, N_CHUNKS, EK)

    deg2 = _sc_degree(dst3)
    deg_col = deg2.T

    u = _tc_scale(x, deg_col)
    agg1 = _sc_aggregate(u, src3, dst3, D)

    W2p = jnp.zeros((H, C_PAD), jnp.float32).at[:, :C].set(W2)
    b2p = jnp.zeros((1, C_PAD), jnp.float32).at[0, :C].set(b2)
    y2 = _tc_mid(agg1, u, deg_col, W1, b1.reshape(1, H), W2p)
    agg2 = _sc_aggregate(y2, src3, dst3, C_PAD)

    out = _tc_combine_final(agg2, y2, deg_col, b2p)
    return out[:, :C]


# speedup vs baseline: 30.5138x; 1.0164x over previous
import functools

import jax
import jax.numpy as jnp
from jax import lax
from jax.experimental import pallas as pl
from jax.experimental.pallas import tpu as pltpu
from jax.experimental.pallas import tpu_sc as plsc

N = 10000
E = 320000
D = 128
H = 128
C = 40
C_PAD = 64

NC = 2
NS = 16
NW = NC * NS
EK = 80
E_PER_W = E // NW
N_CHUNKS = E_PER_W // EK
R_BIG = 632
R_LAST = N - (NS - 1) * R_BIG
DEG_PAD = 10240
DEG_PER_TILE = DEG_PAD // NS

_MESH = dict(core_axis_name="c", subcore_axis_name="s", num_cores=NC,
             num_subcores=NS)


def _sc_degree(dst3):
    @functools.partial(
        pl.kernel,
        out_type=jax.ShapeDtypeStruct((NC * DEG_PAD,), jnp.float32),
        mesh=plsc.VectorSubcoreMesh(**_MESH),
        scratch_types=[
            pltpu.VMEM((N_CHUNKS, EK), jnp.int32),
            pltpu.VMEM((EK,), jnp.float32),
            pltpu.VMEM((DEG_PER_TILE,), jnp.float32),
            pltpu.VMEM_SHARED((DEG_PAD,), jnp.float32),
        ],
    )
    def deg_kernel(dst_hbm, out_hbm, dst_all, ones_v, zbuf, deg_sh):
        cid = lax.axis_index("c")
        sid = lax.axis_index("s")
        w = cid * NS + sid

        for j in range(EK // 16):
            ones_v[pl.ds(j * 16, 16)] = jnp.ones((16,), jnp.float32)
        for j in range(DEG_PER_TILE // 16):
            zbuf[pl.ds(j * 16, 16)] = jnp.zeros((16,), jnp.float32)
        pltpu.sync_copy(zbuf, deg_sh.at[pl.ds(sid * DEG_PER_TILE,
                                              DEG_PER_TILE)])
        pltpu.sync_copy(dst_hbm.at[w], dst_all)
        plsc.subcore_barrier()

        def body(i, carry):
            pltpu.sync_copy(ones_v, deg_sh.at[dst_all.at[i]], add=True)
            return carry

        lax.fori_loop(0, N_CHUNKS, body, 0)
        plsc.subcore_barrier()
        pltpu.sync_copy(
            deg_sh.at[pl.ds(sid * DEG_PER_TILE, DEG_PER_TILE)],
            out_hbm.at[pl.ds(cid * DEG_PAD + sid * DEG_PER_TILE,
                             DEG_PER_TILE)])

    return deg_kernel(dst3).reshape(NC, DEG_PAD)


def _sc_aggregate(y, src3, dst3, f):
    @functools.partial(
        pl.kernel,
        out_type=jax.ShapeDtypeStruct((NC * N, f), jnp.float32),
        mesh=plsc.VectorSubcoreMesh(**_MESH),
        compiler_params=pltpu.CompilerParams(use_tc_tiling_on_sc=False),
        scratch_types=[
            pltpu.VMEM((N_CHUNKS, EK), jnp.int32),
            pltpu.VMEM((N_CHUNKS, EK), jnp.int32),
            pltpu.VMEM((EK, f), jnp.float32),
            pltpu.VMEM((EK, f), jnp.float32),
            pltpu.VMEM((EK, f), jnp.float32),
            pltpu.VMEM_SHARED((N, f), jnp.float32),
            pltpu.SemaphoreType.DMA,
            pltpu.SemaphoreType.DMA,
            pltpu.SemaphoreType.DMA,
            pltpu.SemaphoreType.DMA,
            pltpu.SemaphoreType.DMA,
            pltpu.SemaphoreType.DMA,
        ],
    )
    def agg_kernel(y_hbm, src_hbm, dst_hbm, out_hbm, src_all, dst_all,
                   r0, r1, r2, agg_sh, gs0, gs1, gs2, ss0, ss1, ss2):
        rows = [r0, r1, r2]
        gsem = [gs0, gs1, gs2]
        ssem = [ss0, ss1, ss2]
        cid = lax.axis_index("c")
        sid = lax.axis_index("s")
        w = cid * NS + sid

        r0 = sid * R_BIG

        @pl.when(sid < NS - 1)
        def _():
            pltpu.sync_copy(y_hbm.at[pl.ds(r0, R_BIG)],
                            agg_sh.at[pl.ds(r0, R_BIG)])

        @pl.when(sid == NS - 1)
        def _():
            pltpu.sync_copy(y_hbm.at[pl.ds((NS - 1) * R_BIG, R_LAST)],
                            agg_sh.at[pl.ds((NS - 1) * R_BIG, R_LAST)])

        pltpu.sync_copy(src_hbm.at[w], src_all)
        pltpu.sync_copy(dst_hbm.at[w], dst_all)
        plsc.subcore_barrier()

        def g_issue(k, s):
            pltpu.async_copy(y_hbm.at[src_all.at[k]], rows[s], gsem[s])

        def g_wait(k, s):
            pltpu.make_async_copy(y_hbm.at[src_all.at[k]], rows[s],
                                  gsem[s]).wait()

        def s_issue(k, s):
            pltpu.async_copy(rows[s], agg_sh.at[dst_all.at[k]], ssem[s],
                             add=True)

        def s_wait(k, s):
            pltpu.make_async_copy(rows[s], agg_sh.at[dst_all.at[k]],
                                  ssem[s]).wait()

        for s in range(3):
            g_issue(s, s)

        n_rounds = (N_CHUNKS - 2) // 3

        def body(j, carry):
            k0 = 3 * j
            for s in range(3):
                g_wait(k0 + s, s)
                s_issue(k0 + s, s)
            for s in range(3):
                s_wait(k0 + s, s)
                if s < 2:
                    g_issue(k0 + 3 + s, s)
                else:
                    @pl.when(j < n_rounds - 1)
                    def _():
                        g_issue(k0 + 3 + s, s)
            return carry

        lax.fori_loop(0, n_rounds, body, 0)
        for t, s in ((N_CHUNKS - 2, 0), (N_CHUNKS - 1, 1)):
            g_wait(t, s)
            pltpu.sync_copy(rows[s], agg_sh.at[dst_all.at[t]], add=True)
        plsc.subcore_barrier()

        @pl.when(sid < NS - 1)
        def _():
            pltpu.sync_copy(agg_sh.at[pl.ds(r0, R_BIG)],
                            out_hbm.at[pl.ds(cid * N + r0, R_BIG)])

        @pl.when(sid == NS - 1)
        def _():
            pltpu.sync_copy(
                agg_sh.at[pl.ds((NS - 1) * R_BIG, R_LAST)],
                out_hbm.at[pl.ds(cid * N + (NS - 1) * R_BIG, R_LAST)])

    return agg_kernel(y, src3, dst3).reshape(NC, N, f)


_BLK = 400
_GRID = N // _BLK


def _dinv_block(d_ref):
    return lax.rsqrt(d_ref[:, 0:1] + d_ref[:, 1:2] + 1.0)


def _tc_scale(x, deg_col):
    def body(d_ref, x_ref, u_ref):
        u_ref[...] = x_ref[...] * _dinv_block(d_ref)

    return pl.pallas_call(
        body,
        grid=(_GRID,),
        in_specs=[
            pl.BlockSpec((_BLK, 2), lambda i: (i, 0)),
            pl.BlockSpec((_BLK, D), lambda i: (i, 0)),
        ],
        out_specs=pl.BlockSpec((_BLK, D), lambda i: (i, 0)),
        out_shape=jax.ShapeDtypeStruct((N, D), jnp.float32),
    )(deg_col, x)


def _tc_mid(agg2, u, deg_col, w1, b1_row, w2p):
    def body(a0_ref, a1_ref, u_ref, d_ref, w1_ref, b1_ref, w2_ref, out_ref):
        dinv = _dinv_block(d_ref)
        v = a0_ref[...] + a1_ref[...] - u_ref[...]
        t = jnp.dot(v, w1_ref[...],
                    preferred_element_type=jnp.float32) * dinv + b1_ref[...]
        h = jnp.maximum(t, 0.0)
        out_ref[...] = jnp.dot(h, w2_ref[...],
                               preferred_element_type=jnp.float32) * dinv

    f_out = w2p.shape[1]
    flat = agg2.reshape(NC * N, D)
    return pl.pallas_call(
        body,
        grid=(_GRID,),
        in_specs=[
            pl.BlockSpec((_BLK, D), lambda i: (i, 0)),
            pl.BlockSpec((_BLK, D), lambda i: (i + _GRID, 0)),
            pl.BlockSpec((_BLK, D), lambda i: (i, 0)),
            pl.BlockSpec((_BLK, 2), lambda i: (i, 0)),
            pl.BlockSpec((D, D), lambda i: (0, 0)),
            pl.BlockSpec((1, D), lambda i: (0, 0)),
            pl.BlockSpec((D, f_out), lambda i: (0, 0)),
        ],
        out_specs=pl.BlockSpec((_BLK, f_out), lambda i: (i, 0)),
        out_shape=jax.ShapeDtypeStruct((N, f_out), jnp.float32),
    )(flat, flat, u, deg_col, w1, b1_row, w2p)


def _tc_combine_final(agg2, y, deg_col, b_row):
    def body(a0_ref, a1_ref, y_ref, d_ref, b_ref, out_ref):
        out_ref[...] = ((a0_ref[...] + a1_ref[...] - y_ref[...])
                        * _dinv_block(d_ref) + b_ref[...])

    f = y.shape[1]
    flat = agg2.reshape(NC * N, f)
    return pl.pallas_call(
        body,
        grid=(_GRID,),
        in_specs=[
            pl.BlockSpec((_BLK, f), lambda i: (i, 0)),
            pl.BlockSpec((_BLK, f), lambda i: (i + _GRID, 0)),
            pl.BlockSpec((_BLK, f), lambda i: (i, 0)),
            pl.BlockSpec((_BLK, 2), lambda i: (i, 0)),
            pl.BlockSpec((1, f), lambda i: (0, 0)),
        ],
        out_specs=pl.BlockSpec((_BLK, f), lambda i: (i, 0)),
        out_shape=jax.ShapeDtypeStruct((N, f), jnp.float32),
    )(flat, flat, y, deg_col, b_row)


def kernel(x, edge_index, W1, b1, W2, b2):
    src3 = edge_index[0].reshape(NW, N_CHUNKS, EK)
    dst3 = edge_index[1].reshape(NW, N_CHUNKS, EK)

    deg2 = _sc_degree(dst3)
    deg_col = deg2.T

    u = _tc_scale(x, deg_col)
    agg1 = _sc_aggregate(u, src3, dst3, D)

    W2p = jnp.zeros((H, C_PAD), jnp.float32).at[:, :C].set(W2)
    b2p = jnp.zeros((1, C_PAD), jnp.float32).at[0, :C].set(b2)
    y2 = _tc_mid(agg1, u, deg_col, W1, b1.reshape(1, H), W2p)
    agg2 = _sc_aggregate(y2, src3, dst3, C_PAD)

    out = _tc_combine_final(agg2, y2, deg_col, b2p)
    return out[:, :C]


# speedup vs baseline: 33.5256x; 1.0987x over previous
import functools

import jax
import jax.numpy as jnp
from jax import lax
from jax.experimental import pallas as pl
from jax.experimental.pallas import tpu as pltpu
from jax.experimental.pallas import tpu_sc as plsc

N = 10000
E = 320000
D = 128
H = 128
C = 40
C_PAD = 64

NC = 2
NS = 16
NW = NC * NS
EK = 80
E_PER_W = E // NW
N_CHUNKS = E_PER_W // EK
R_BIG = 632
R_LAST = N - (NS - 1) * R_BIG
DEG_PAD = 10240
DEG_PER_TILE = DEG_PAD // NS

_MESH = dict(core_axis_name="c", subcore_axis_name="s", num_cores=NC,
             num_subcores=NS)


def _sc_degree(dst3):
    @functools.partial(
        pl.kernel,
        out_type=jax.ShapeDtypeStruct((NC * DEG_PAD,), jnp.float32),
        mesh=plsc.VectorSubcoreMesh(**_MESH),
        scratch_types=[
            pltpu.VMEM((N_CHUNKS, EK), jnp.int32),
            pltpu.VMEM((EK,), jnp.float32),
            pltpu.VMEM((DEG_PER_TILE,), jnp.float32),
            pltpu.VMEM_SHARED((DEG_PAD,), jnp.float32),
        ],
    )
    def deg_kernel(dst_hbm, out_hbm, dst_all, ones_v, zbuf, deg_sh):
        cid = lax.axis_index("c")
        sid = lax.axis_index("s")
        w = cid * NS + sid

        for j in range(EK // 16):
            ones_v[pl.ds(j * 16, 16)] = jnp.ones((16,), jnp.float32)
        for j in range(DEG_PER_TILE // 16):
            zbuf[pl.ds(j * 16, 16)] = jnp.zeros((16,), jnp.float32)
        pltpu.sync_copy(zbuf, deg_sh.at[pl.ds(sid * DEG_PER_TILE,
                                              DEG_PER_TILE)])
        pltpu.sync_copy(dst_hbm.at[w], dst_all)
        plsc.subcore_barrier()

        def body(i, carry):
            pltpu.sync_copy(ones_v, deg_sh.at[dst_all.at[i]], add=True)
            return carry

        lax.fori_loop(0, N_CHUNKS, body, 0)
        plsc.subcore_barrier()
        pltpu.sync_copy(
            deg_sh.at[pl.ds(sid * DEG_PER_TILE, DEG_PER_TILE)],
            out_hbm.at[pl.ds(cid * DEG_PAD + sid * DEG_PER_TILE,
                             DEG_PER_TILE)])

    return deg_kernel(dst3).reshape(NC, DEG_PAD)


def _sc_aggregate(y, src3, dst3, f):
    @functools.partial(
        pl.kernel,
        out_type=jax.ShapeDtypeStruct((NC * N, f), jnp.float32),
        mesh=plsc.VectorSubcoreMesh(**_MESH),
        compiler_params=pltpu.CompilerParams(use_tc_tiling_on_sc=False),
        scratch_types=[
            pltpu.VMEM((N_CHUNKS, EK), jnp.int32),
            pltpu.VMEM((N_CHUNKS, EK), jnp.int32),
            pltpu.VMEM((EK, f), jnp.float32),
            pltpu.VMEM((EK, f), jnp.float32),
            pltpu.VMEM((EK, f), jnp.float32),
            pltpu.VMEM_SHARED((N, f), jnp.float32),
            pltpu.SemaphoreType.DMA,
            pltpu.SemaphoreType.DMA,
            pltpu.SemaphoreType.DMA,
            pltpu.SemaphoreType.DMA,
            pltpu.SemaphoreType.DMA,
            pltpu.SemaphoreType.DMA,
        ],
    )
    def agg_kernel(y_hbm, src_hbm, dst_hbm, out_hbm, src_all, dst_all,
                   r0, r1, r2, agg_sh, gs0, gs1, gs2, ss0, ss1, ss2):
        rows = [r0, r1, r2]
        gsem = [gs0, gs1, gs2]
        ssem = [ss0, ss1, ss2]
        cid = lax.axis_index("c")
        sid = lax.axis_index("s")
        w = cid * NS + sid

        r0 = sid * R_BIG

        @pl.when(sid < NS - 1)
        def _():
            pltpu.sync_copy(y_hbm.at[pl.ds(r0, R_BIG)],
                            agg_sh.at[pl.ds(r0, R_BIG)])

        @pl.when(sid == NS - 1)
        def _():
            pltpu.sync_copy(y_hbm.at[pl.ds((NS - 1) * R_BIG, R_LAST)],
                            agg_sh.at[pl.ds((NS - 1) * R_BIG, R_LAST)])

        pltpu.sync_copy(src_hbm.at[w], src_all)
        pltpu.sync_copy(dst_hbm.at[w], dst_all)
        plsc.subcore_barrier()

        def g_issue(k, s):
            pltpu.async_copy(y_hbm.at[src_all.at[k]], rows[s], gsem[s])

        def g_wait(k, s):
            pltpu.make_async_copy(y_hbm.at[src_all.at[k]], rows[s],
                                  gsem[s]).wait()

        def s_issue(k, s):
            pltpu.async_copy(rows[s], agg_sh.at[dst_all.at[k]], ssem[s],
                             add=True)

        def s_wait(k, s):
            pltpu.make_async_copy(rows[s], agg_sh.at[dst_all.at[k]],
                                  ssem[s]).wait()

        for s in range(3):
            g_issue(s, s)

        n_rounds = (N_CHUNKS - 2) // 3

        def body(j, carry):
            k0 = 3 * j
            for s in range(3):
                g_wait(k0 + s, s)
                s_issue(k0 + s, s)
            for s in range(3):
                s_wait(k0 + s, s)
                if s < 2:
                    g_issue(k0 + 3 + s, s)
                else:
                    @pl.when(j < n_rounds - 1)
                    def _():
                        g_issue(k0 + 3 + s, s)
            return carry

        lax.fori_loop(0, n_rounds, body, 0)
        for t, s in ((N_CHUNKS - 2, 0), (N_CHUNKS - 1, 1)):
            g_wait(t, s)
            pltpu.sync_copy(rows[s], agg_sh.at[dst_all.at[t]], add=True)
        plsc.subcore_barrier()

        @pl.when(sid < NS - 1)
        def _():
            pltpu.sync_copy(agg_sh.at[pl.ds(r0, R_BIG)],
                            out_hbm.at[pl.ds(cid * N + r0, R_BIG)])

        @pl.when(sid == NS - 1)
        def _():
            pltpu.sync_copy(
                agg_sh.at[pl.ds((NS - 1) * R_BIG, R_LAST)],
                out_hbm.at[pl.ds(cid * N + (NS - 1) * R_BIG, R_LAST)])

    return agg_kernel(y, src3, dst3).reshape(NC, N, f)


_BLK = 2000
_GRID = N // _BLK


def _dinv_block(d_ref):
    return lax.rsqrt(d_ref[:, 0:1] + d_ref[:, 1:2] + 1.0)


def _tc_scale(x, deg_col):
    def body(d_ref, x_ref, u_ref):
        u_ref[...] = x_ref[...] * _dinv_block(d_ref)

    return pl.pallas_call(
        body,
        grid=(_GRID,),
        in_specs=[
            pl.BlockSpec((_BLK, 2), lambda i: (i, 0)),
            pl.BlockSpec((_BLK, D), lambda i: (i, 0)),
        ],
        out_specs=pl.BlockSpec((_BLK, D), lambda i: (i, 0)),
        out_shape=jax.ShapeDtypeStruct((N, D), jnp.float32),
    )(deg_col, x)


def _tc_mid(agg2, u, deg_col, w1, b1_row, w2p):
    def body(a0_ref, a1_ref, u_ref, d_ref, w1_ref, b1_ref, w2_ref, out_ref):
        dinv = _dinv_block(d_ref)
        v = a0_ref[...] + a1_ref[...] - u_ref[...]
        t = jnp.dot(v, w1_ref[...],
                    preferred_element_type=jnp.float32) * dinv + b1_ref[...]
        h = jnp.maximum(t, 0.0)
        out_ref[...] = jnp.dot(h, w2_ref[...],
                               preferred_element_type=jnp.float32) * dinv

    f_out = w2p.shape[1]
    flat = agg2.reshape(NC * N, D)
    return pl.pallas_call(
        body,
        grid=(_GRID,),
        in_specs=[
            pl.BlockSpec((_BLK, D), lambda i: (i, 0)),
            pl.BlockSpec((_BLK, D), lambda i: (i + _GRID, 0)),
            pl.BlockSpec((_BLK, D), lambda i: (i, 0)),
            pl.BlockSpec((_BLK, 2), lambda i: (i, 0)),
            pl.BlockSpec((D, D), lambda i: (0, 0)),
            pl.BlockSpec((1, D), lambda i: (0, 0)),
            pl.BlockSpec((D, f_out), lambda i: (0, 0)),
        ],
        out_specs=pl.BlockSpec((_BLK, f_out), lambda i: (i, 0)),
        out_shape=jax.ShapeDtypeStruct((N, f_out), jnp.float32),
    )(flat, flat, u, deg_col, w1, b1_row, w2p)


def _tc_combine_final(agg2, y, deg_col, b_row):
    def body(a0_ref, a1_ref, y_ref, d_ref, b_ref, out_ref):
        out_ref[...] = ((a0_ref[...] + a1_ref[...] - y_ref[...])
                        * _dinv_block(d_ref) + b_ref[...])

    f = y.shape[1]
    flat = agg2.reshape(NC * N, f)
    return pl.pallas_call(
        body,
        grid=(_GRID,),
        in_specs=[
            pl.BlockSpec((_BLK, f), lambda i: (i, 0)),
            pl.BlockSpec((_BLK, f), lambda i: (i + _GRID, 0)),
            pl.BlockSpec((_BLK, f), lambda i: (i, 0)),
            pl.BlockSpec((_BLK, 2), lambda i: (i, 0)),
            pl.BlockSpec((1, f), lambda i: (0, 0)),
        ],
        out_specs=pl.BlockSpec((_BLK, f), lambda i: (i, 0)),
        out_shape=jax.ShapeDtypeStruct((N, f), jnp.float32),
    )(flat, flat, y, deg_col, b_row)


def kernel(x, edge_index, W1, b1, W2, b2):
    src3 = edge_index[0].reshape(NW, N_CHUNKS, EK)
    dst3 = edge_index[1].reshape(NW, N_CHUNKS, EK)

    deg2 = _sc_degree(dst3)
    deg_col = deg2.T

    u = _tc_scale(x, deg_col)
    agg1 = _sc_aggregate(u, src3, dst3, D)

    W2p = jnp.zeros((H, C_PAD), jnp.float32).at[:, :C].set(W2)
    b2p = jnp.zeros((1, C_PAD), jnp.float32).at[0, :C].set(b2)
    y2 = _tc_mid(agg1, u, deg_col, W1, b1.reshape(1, H), W2p)
    agg2 = _sc_aggregate(y2, src3, dst3, C_PAD)

    out = _tc_combine_final(agg2, y2, deg_col, b2p)
    return out[:, :C]


# speedup vs baseline: 36.5510x; 1.0902x over previous
import functools

import jax
import jax.numpy as jnp
from jax import lax
from jax.experimental import pallas as pl
from jax.experimental.pallas import tpu as pltpu
from jax.experimental.pallas import tpu_sc as plsc

N = 10000
E = 320000
D = 128
H = 128
C = 40
C_PAD = 64

NC = 2
NS = 16
NW = NC * NS
EK = 80
E_PER_W = E // NW
N_CHUNKS = E_PER_W // EK
R_BIG = 632
R_LAST = N - (NS - 1) * R_BIG
DEG_PAD = 10240
DEG_PER_TILE = DEG_PAD // NS

_MESH = dict(core_axis_name="c", subcore_axis_name="s", num_cores=NC,
             num_subcores=NS)


def _sc_degree(edge4):
    @functools.partial(
        pl.kernel,
        out_type=jax.ShapeDtypeStruct((NC * DEG_PAD,), jnp.float32),
        mesh=plsc.VectorSubcoreMesh(**_MESH),
        scratch_types=[
            pltpu.VMEM((N_CHUNKS, EK), jnp.int32),
            pltpu.VMEM((EK,), jnp.float32),
            pltpu.VMEM((DEG_PER_TILE,), jnp.float32),
            pltpu.VMEM_SHARED((DEG_PAD,), jnp.float32),
        ],
    )
    def deg_kernel(dst_hbm, out_hbm, dst_all, ones_v, zbuf, deg_sh):
        cid = lax.axis_index("c")
        sid = lax.axis_index("s")
        w = cid * NS + sid

        for j in range(EK // 16):
            ones_v[pl.ds(j * 16, 16)] = jnp.ones((16,), jnp.float32)
        for j in range(DEG_PER_TILE // 16):
            zbuf[pl.ds(j * 16, 16)] = jnp.zeros((16,), jnp.float32)
        pltpu.sync_copy(zbuf, deg_sh.at[pl.ds(sid * DEG_PER_TILE,
                                              DEG_PER_TILE)])
        pltpu.sync_copy(dst_hbm.at[1, w], dst_all)
        plsc.subcore_barrier()

        def body(i, carry):
            pltpu.sync_copy(ones_v, deg_sh.at[dst_all.at[i]], add=True)
            return carry

        lax.fori_loop(0, N_CHUNKS, body, 0)
        plsc.subcore_barrier()
        pltpu.sync_copy(
            deg_sh.at[pl.ds(sid * DEG_PER_TILE, DEG_PER_TILE)],
            out_hbm.at[pl.ds(cid * DEG_PAD + sid * DEG_PER_TILE,
                             DEG_PER_TILE)])

    return deg_kernel(edge4).reshape(NC, DEG_PAD)


def _sc_aggregate(y, edge4, f):
    @functools.partial(
        pl.kernel,
        out_type=jax.ShapeDtypeStruct((NC * N, f), jnp.float32),
        mesh=plsc.VectorSubcoreMesh(**_MESH),
        compiler_params=pltpu.CompilerParams(use_tc_tiling_on_sc=False),
        scratch_types=[
            pltpu.VMEM((N_CHUNKS, EK), jnp.int32),
            pltpu.VMEM((N_CHUNKS, EK), jnp.int32),
            pltpu.VMEM((EK, f), jnp.float32),
            pltpu.VMEM((EK, f), jnp.float32),
            pltpu.VMEM((EK, f), jnp.float32),
            pltpu.VMEM_SHARED((N, f), jnp.float32),
            pltpu.SemaphoreType.DMA,
            pltpu.SemaphoreType.DMA,
            pltpu.SemaphoreType.DMA,
            pltpu.SemaphoreType.DMA,
            pltpu.SemaphoreType.DMA,
            pltpu.SemaphoreType.DMA,
        ],
    )
    def agg_kernel(y_hbm, edge_hbm, out_hbm, src_all, dst_all,
                   rb0, rb1, rb2, agg_sh, gs0, gs1, gs2, ss0, ss1, ss2):
        rows = [rb0, rb1, rb2]
        gsem = [gs0, gs1, gs2]
        ssem = [ss0, ss1, ss2]
        cid = lax.axis_index("c")
        sid = lax.axis_index("s")
        w = cid * NS + sid
        r0 = sid * R_BIG

        pltpu.sync_copy(edge_hbm.at[0, w], src_all)
        pltpu.sync_copy(edge_hbm.at[1, w], dst_all)

        def g_issue(k, s):
            pltpu.async_copy(y_hbm.at[src_all.at[k]], rows[s], gsem[s])

        def g_wait(k, s):
            pltpu.make_async_copy(y_hbm.at[src_all.at[k]], rows[s],
                                  gsem[s]).wait()

        def s_issue(k, s):
            pltpu.async_copy(rows[s], agg_sh.at[dst_all.at[k]], ssem[s],
                             add=True)

        def s_wait(k, s):
            pltpu.make_async_copy(rows[s], agg_sh.at[dst_all.at[k]],
                                  ssem[s]).wait()

        for s in range(3):
            g_issue(s, s)

        @pl.when(sid < NS - 1)
        def _():
            pltpu.sync_copy(y_hbm.at[pl.ds(r0, R_BIG)],
                            agg_sh.at[pl.ds(r0, R_BIG)])

        @pl.when(sid == NS - 1)
        def _():
            pltpu.sync_copy(y_hbm.at[pl.ds((NS - 1) * R_BIG, R_LAST)],
                            agg_sh.at[pl.ds((NS - 1) * R_BIG, R_LAST)])

        plsc.subcore_barrier()

        n_rounds = (N_CHUNKS - 2) // 3

        def body(j, carry):
            k0 = 3 * j
            for s in range(3):
                g_wait(k0 + s, s)
                s_issue(k0 + s, s)
            for s in range(3):
                s_wait(k0 + s, s)
                if s < 2:
                    g_issue(k0 + 3 + s, s)
                else:
                    @pl.when(j < n_rounds - 1)
                    def _():
                        g_issue(k0 + 3 + s, s)
            return carry

        lax.fori_loop(0, n_rounds, body, 0)
        for t, s in ((N_CHUNKS - 2, 0), (N_CHUNKS - 1, 1)):
            g_wait(t, s)
            pltpu.sync_copy(rows[s], agg_sh.at[dst_all.at[t]], add=True)
        plsc.subcore_barrier()

        @pl.when(sid < NS - 1)
        def _():
            pltpu.sync_copy(agg_sh.at[pl.ds(r0, R_BIG)],
                            out_hbm.at[pl.ds(cid * N + r0, R_BIG)])

        @pl.when(sid == NS - 1)
        def _():
            pltpu.sync_copy(
                agg_sh.at[pl.ds((NS - 1) * R_BIG, R_LAST)],
                out_hbm.at[pl.ds(cid * N + (NS - 1) * R_BIG, R_LAST)])

    return agg_kernel(y, edge4).reshape(NC, N, f)


_BLK = 2000
_GRID = N // _BLK


def _dinv_block(d_ref):
    return lax.rsqrt(d_ref[:, 0:1] + d_ref[:, 1:2] + 1.0)


def _tc_scale(x, deg_col):
    def body(d_ref, x_ref, u_ref):
        u_ref[...] = x_ref[...] * _dinv_block(d_ref)

    return pl.pallas_call(
        body,
        grid=(_GRID,),
        in_specs=[
            pl.BlockSpec((_BLK, 2), lambda i: (i, 0)),
            pl.BlockSpec((_BLK, D), lambda i: (i, 0)),
        ],
        out_specs=pl.BlockSpec((_BLK, D), lambda i: (i, 0)),
        out_shape=jax.ShapeDtypeStruct((N, D), jnp.float32),
    )(deg_col, x)


def _tc_mid(agg2, u, deg_col, w1, b1_row, w2p):
    def body(a0_ref, a1_ref, u_ref, d_ref, w1_ref, b1_ref, w2_ref, out_ref):
        dinv = _dinv_block(d_ref)
        v = a0_ref[...] + a1_ref[...] - u_ref[...]
        t = jnp.dot(v, w1_ref[...],
                    preferred_element_type=jnp.float32) * dinv + b1_ref[...]
        h = jnp.maximum(t, 0.0)
        out_ref[...] = jnp.dot(h, w2_ref[...],
                               preferred_element_type=jnp.float32) * dinv

    f_out = w2p.shape[1]
    flat = agg2.reshape(NC * N, D)
    return pl.pallas_call(
        body,
        grid=(_GRID,),
        in_specs=[
            pl.BlockSpec((_BLK, D), lambda i: (i, 0)),
            pl.BlockSpec((_BLK, D), lambda i: (i + _GRID, 0)),
            pl.BlockSpec((_BLK, D), lambda i: (i, 0)),
            pl.BlockSpec((_BLK, 2), lambda i: (i, 0)),
            pl.BlockSpec((D, D), lambda i: (0, 0)),
            pl.BlockSpec((1, D), lambda i: (0, 0)),
            pl.BlockSpec((D, f_out), lambda i: (0, 0)),
        ],
        out_specs=pl.BlockSpec((_BLK, f_out), lambda i: (i, 0)),
        out_shape=jax.ShapeDtypeStruct((N, f_out), jnp.float32),
    )(flat, flat, u, deg_col, w1, b1_row, w2p)


def _tc_combine_final(agg2, y, deg_col, b_row):
    def body(a0_ref, a1_ref, y_ref, d_ref, b_ref, out_ref):
        out_ref[...] = ((a0_ref[...] + a1_ref[...] - y_ref[...])
                        * _dinv_block(d_ref) + b_ref[...])

    f = y.shape[1]
    flat = agg2.reshape(NC * N, f)
    return pl.pallas_call(
        body,
        grid=(_GRID,),
        in_specs=[
            pl.BlockSpec((_BLK, f), lambda i: (i, 0)),
            pl.BlockSpec((_BLK, f), lambda i: (i + _GRID, 0)),
            pl.BlockSpec((_BLK, f), lambda i: (i, 0)),
            pl.BlockSpec((_BLK, 2), lambda i: (i, 0)),
            pl.BlockSpec((1, f), lambda i: (0, 0)),
        ],
        out_specs=pl.BlockSpec((_BLK, f), lambda i: (i, 0)),
        out_shape=jax.ShapeDtypeStruct((N, f), jnp.float32),
    )(flat, flat, y, deg_col, b_row)


def kernel(x, edge_index, W1, b1, W2, b2):
    edge4 = edge_index.reshape(2, NW, N_CHUNKS, EK)

    deg2 = _sc_degree(edge4)
    deg_col = deg2.T

    u = _tc_scale(x, deg_col)
    agg1 = _sc_aggregate(u, edge4, D)

    y2 = _tc_mid(agg1, u, deg_col, W1, b1.reshape(1, H), W2)
    agg2 = _sc_aggregate(y2, edge4, C)

    return _tc_combine_final(agg2, y2, deg_col, b2.reshape(1, C))


# speedup vs baseline: 37.2607x; 1.0194x over previous
import functools

import jax
import jax.numpy as jnp
from jax import lax
from jax.experimental import pallas as pl
from jax.experimental.pallas import tpu as pltpu
from jax.experimental.pallas import tpu_sc as plsc

N = 10000
E = 320000
D = 128
H = 128
C = 40
C_PAD = 64

NC = 2
NS = 16
NW = NC * NS
EK = 80
E_PER_W = E // NW
N_CHUNKS = E_PER_W // EK
R_BIG = 632
R_LAST = N - (NS - 1) * R_BIG
DEG_PAD = 10240
DEG_PER_TILE = DEG_PAD // NS

_MESH = dict(core_axis_name="c", subcore_axis_name="s", num_cores=NC,
             num_subcores=NS)


def _sc_degree(edge4):
    @functools.partial(
        pl.kernel,
        out_type=jax.ShapeDtypeStruct((NC * DEG_PAD,), jnp.float32),
        mesh=plsc.VectorSubcoreMesh(**_MESH),
        scratch_types=[
            pltpu.VMEM((N_CHUNKS, EK), jnp.int32),
            pltpu.VMEM((EK,), jnp.float32),
            pltpu.VMEM((DEG_PER_TILE,), jnp.float32),
            pltpu.VMEM_SHARED((DEG_PAD,), jnp.float32),
            pltpu.SemaphoreType.DMA,
            pltpu.SemaphoreType.DMA,
        ],
    )
    def deg_kernel(dst_hbm, out_hbm, dst_all, ones_v, zbuf, deg_sh,
                   dsem0, dsem1):
        cid = lax.axis_index("c")
        sid = lax.axis_index("s")
        w = cid * NS + sid

        for j in range(EK // 16):
            ones_v[pl.ds(j * 16, 16)] = jnp.ones((16,), jnp.float32)
        for j in range(DEG_PER_TILE // 16):
            zbuf[pl.ds(j * 16, 16)] = jnp.zeros((16,), jnp.float32)
        pltpu.sync_copy(zbuf, deg_sh.at[pl.ds(sid * DEG_PER_TILE,
                                              DEG_PER_TILE)])
        pltpu.sync_copy(dst_hbm.at[1, w], dst_all)
        plsc.subcore_barrier()

        def s_issue(k, sem):
            pltpu.async_copy(ones_v, deg_sh.at[dst_all.at[k]], sem, add=True)

        def s_wait(k, sem):
            pltpu.make_async_copy(ones_v, deg_sh.at[dst_all.at[k]],
                                  sem).wait()

        s_issue(0, dsem0)

        def body(j, carry):
            k0 = 2 * j
            s_issue(k0 + 1, dsem1)
            s_wait(k0, dsem0)
            s_issue(k0 + 2, dsem0)
            s_wait(k0 + 1, dsem1)
            return carry

        lax.fori_loop(0, (N_CHUNKS - 1) // 2, body, 0)
        s_wait(N_CHUNKS - 1, dsem0)
        plsc.subcore_barrier()
        pltpu.sync_copy(
            deg_sh.at[pl.ds(sid * DEG_PER_TILE, DEG_PER_TILE)],
            out_hbm.at[pl.ds(cid * DEG_PAD + sid * DEG_PER_TILE,
                             DEG_PER_TILE)])

    return deg_kernel(edge4).reshape(NC, DEG_PAD)


def _sc_aggregate(y, edge4, f):
    @functools.partial(
        pl.kernel,
        out_type=jax.ShapeDtypeStruct((NC * N, f), jnp.float32),
        mesh=plsc.VectorSubcoreMesh(**_MESH),
        compiler_params=pltpu.CompilerParams(use_tc_tiling_on_sc=False),
        scratch_types=[
            pltpu.VMEM((N_CHUNKS, EK), jnp.int32),
            pltpu.VMEM((N_CHUNKS, EK), jnp.int32),
            pltpu.VMEM((EK, f), jnp.float32),
            pltpu.VMEM((EK, f), jnp.float32),
            pltpu.VMEM((EK, f), jnp.float32),
            pltpu.VMEM_SHARED((N, f), jnp.float32),
            pltpu.SemaphoreType.DMA,
            pltpu.SemaphoreType.DMA,
            pltpu.SemaphoreType.DMA,
            pltpu.SemaphoreType.DMA,
            pltpu.SemaphoreType.DMA,
            pltpu.SemaphoreType.DMA,
        ],
    )
    def agg_kernel(y_hbm, edge_hbm, out_hbm, src_all, dst_all,
                   rb0, rb1, rb2, agg_sh, gs0, gs1, gs2, ss0, ss1, ss2):
        rows = [rb0, rb1, rb2]
        gsem = [gs0, gs1, gs2]
        ssem = [ss0, ss1, ss2]
        cid = lax.axis_index("c")
        sid = lax.axis_index("s")
        w = cid * NS + sid
        r0 = sid * R_BIG

        pltpu.sync_copy(edge_hbm.at[0, w], src_all)
        pltpu.sync_copy(edge_hbm.at[1, w], dst_all)

        def g_issue(k, s):
            pltpu.async_copy(y_hbm.at[src_all.at[k]], rows[s], gsem[s])

        def g_wait(k, s):
            pltpu.make_async_copy(y_hbm.at[src_all.at[k]], rows[s],
                                  gsem[s]).wait()

        def s_issue(k, s):
            pltpu.async_copy(rows[s], agg_sh.at[dst_all.at[k]], ssem[s],
                             add=True)

        def s_wait(k, s):
            pltpu.make_async_copy(rows[s], agg_sh.at[dst_all.at[k]],
                                  ssem[s]).wait()

        for s in range(3):
            g_issue(s, s)

        @pl.when(sid < NS - 1)
        def _():
            pltpu.sync_copy(y_hbm.at[pl.ds(r0, R_BIG)],
                            agg_sh.at[pl.ds(r0, R_BIG)])

        @pl.when(sid == NS - 1)
        def _():
            pltpu.sync_copy(y_hbm.at[pl.ds((NS - 1) * R_BIG, R_LAST)],
                            agg_sh.at[pl.ds((NS - 1) * R_BIG, R_LAST)])

        plsc.subcore_barrier()

        n_rounds = (N_CHUNKS - 2) // 3

        def body(j, carry):
            k0 = 3 * j
            for s in range(3):
                g_wait(k0 + s, s)
                s_issue(k0 + s, s)
            for s in range(3):
                s_wait(k0 + s, s)
                if s < 2:
                    g_issue(k0 + 3 + s, s)
                else:
                    @pl.when(j < n_rounds - 1)
                    def _():
                        g_issue(k0 + 3 + s, s)
            return carry

        lax.fori_loop(0, n_rounds, body, 0)
        for t, s in ((N_CHUNKS - 2, 0), (N_CHUNKS - 1, 1)):
            g_wait(t, s)
            pltpu.sync_copy(rows[s], agg_sh.at[dst_all.at[t]], add=True)
        plsc.subcore_barrier()

        @pl.when(sid < NS - 1)
        def _():
            pltpu.sync_copy(agg_sh.at[pl.ds(r0, R_BIG)],
                            out_hbm.at[pl.ds(cid * N + r0, R_BIG)])

        @pl.when(sid == NS - 1)
        def _():
            pltpu.sync_copy(
                agg_sh.at[pl.ds((NS - 1) * R_BIG, R_LAST)],
                out_hbm.at[pl.ds(cid * N + (NS - 1) * R_BIG, R_LAST)])

    return agg_kernel(y, edge4)


_BLK = 2000
_GRID = N // _BLK


def _dinv_block(d_ref):
    return lax.rsqrt(d_ref[:, 0:1] + d_ref[:, 1:2] + 1.0)


def _tc_scale(x, deg_col):
    def body(d_ref, x_ref, u_ref):
        u_ref[...] = x_ref[...] * _dinv_block(d_ref)

    return pl.pallas_call(
        body,
        grid=(_GRID,),
        in_specs=[
            pl.BlockSpec((_BLK, 2), lambda i: (i, 0)),
            pl.BlockSpec((_BLK, D), lambda i: (i, 0)),
        ],
        out_specs=pl.BlockSpec((_BLK, D), lambda i: (i, 0)),
        out_shape=jax.ShapeDtypeStruct((N, D), jnp.float32),
    )(deg_col, x)


def _tc_mid(agg2, u, deg_col, w1, b1_row, w2p):
    def body(a0_ref, a1_ref, u_ref, d_ref, w1_ref, b1_ref, w2_ref, out_ref):
        dinv = _dinv_block(d_ref)
        v = a0_ref[...] + a1_ref[...] - u_ref[...]
        t = jnp.dot(v, w1_ref[...],
                    preferred_element_type=jnp.float32) * dinv + b1_ref[...]
        h = jnp.maximum(t, 0.0)
        out_ref[...] = jnp.dot(h, w2_ref[...],
                               preferred_element_type=jnp.float32) * dinv

    f_out = w2p.shape[1]
    flat = agg2
    return pl.pallas_call(
        body,
        grid=(_GRID,),
        in_specs=[
            pl.BlockSpec((_BLK, D), lambda i: (i, 0)),
            pl.BlockSpec((_BLK, D), lambda i: (i + _GRID, 0)),
            pl.BlockSpec((_BLK, D), lambda i: (i, 0)),
            pl.BlockSpec((_BLK, 2), lambda i: (i, 0)),
            pl.BlockSpec((D, D), lambda i: (0, 0)),
            pl.BlockSpec((1, D), lambda i: (0, 0)),
            pl.BlockSpec((D, f_out), lambda i: (0, 0)),
        ],
        out_specs=pl.BlockSpec((_BLK, f_out), lambda i: (i, 0)),
        out_shape=jax.ShapeDtypeStruct((N, f_out), jnp.float32),
    )(flat, flat, u, deg_col, w1, b1_row, w2p)


def _tc_combine_final(agg2, y, deg_col, b_row):
    def body(a0_ref, a1_ref, y_ref, d_ref, b_ref, out_ref):
        out_ref[...] = ((a0_ref[...] + a1_ref[...] - y_ref[...])
                        * _dinv_block(d_ref) + b_ref[...])

    f = y.shape[1]
    flat = agg2
    return pl.pallas_call(
        body,
        grid=(_GRID,),
        in_specs=[
            pl.BlockSpec((_BLK, f), lambda i: (i, 0)),
            pl.BlockSpec((_BLK, f), lambda i: (i + _GRID, 0)),
            pl.BlockSpec((_BLK, f), lambda i: (i, 0)),
            pl.BlockSpec((_BLK, 2), lambda i: (i, 0)),
            pl.BlockSpec((1, f), lambda i: (0, 0)),
        ],
        out_specs=pl.BlockSpec((_BLK, f), lambda i: (i, 0)),
        out_shape=jax.ShapeDtypeStruct((N, f), jnp.float32),
    )(flat, flat, y, deg_col, b_row)


def kernel(x, edge_index, W1, b1, W2, b2):
    edge4 = edge_index.reshape(2, NW, N_CHUNKS, EK)

    deg2 = _sc_degree(edge4)
    deg_col = deg2.T

    u = _tc_scale(x, deg_col)
    agg1 = _sc_aggregate(u, edge4, D)

    y2 = _tc_mid(agg1, u, deg_col, W1, b1.reshape(1, H), W2)
    agg2 = _sc_aggregate(y2, edge4, C)

    return _tc_combine_final(agg2, y2, deg_col, b2.reshape(1, C))


# speedup vs baseline: 39.3964x; 1.0573x over previous
import functools

import jax
import jax.numpy as jnp
from jax import lax
from jax.experimental import pallas as pl
from jax.experimental.pallas import tpu as pltpu
from jax.experimental.pallas import tpu_sc as plsc

N = 10000
E = 320000
D = 128
H = 128
C = 40
C_PAD = 64

NC = 2
NS = 16
NW = NC * NS
EK = 80
E_PER_W = E // NW
N_CHUNKS = E_PER_W // EK
R_BIG = 632
R_LAST = N - (NS - 1) * R_BIG
DEG_PAD = 10240
DEG_PER_TILE = DEG_PAD // NS

_MESH = dict(core_axis_name="c", subcore_axis_name="s", num_cores=NC,
             num_subcores=NS)


def _sc_degree(edge4):
    @functools.partial(
        pl.kernel,
        out_type=jax.ShapeDtypeStruct((NC * DEG_PAD,), jnp.float32),
        mesh=plsc.VectorSubcoreMesh(**_MESH),
        scratch_types=[
            pltpu.VMEM((N_CHUNKS, EK), jnp.int32),
            pltpu.VMEM((EK,), jnp.float32),
            pltpu.VMEM((DEG_PER_TILE,), jnp.float32),
            pltpu.VMEM_SHARED((DEG_PAD,), jnp.float32),
            pltpu.SemaphoreType.DMA,
            pltpu.SemaphoreType.DMA,
        ],
    )
    def deg_kernel(dst_hbm, out_hbm, dst_all, ones_v, zbuf, deg_sh,
                   dsem0, dsem1):
        cid = lax.axis_index("c")
        sid = lax.axis_index("s")
        w = cid * NS + sid

        for j in range(EK // 16):
            ones_v[pl.ds(j * 16, 16)] = jnp.ones((16,), jnp.float32)
        for j in range(DEG_PER_TILE // 16):
            zbuf[pl.ds(j * 16, 16)] = jnp.zeros((16,), jnp.float32)
        pltpu.sync_copy(zbuf, deg_sh.at[pl.ds(sid * DEG_PER_TILE,
                                              DEG_PER_TILE)])
        pltpu.sync_copy(dst_hbm.at[1, w], dst_all)
        plsc.subcore_barrier()

        def s_issue(k, sem):
            pltpu.async_copy(ones_v, deg_sh.at[dst_all.at[k]], sem, add=True)

        def s_wait(k, sem):
            pltpu.make_async_copy(ones_v, deg_sh.at[dst_all.at[k]],
                                  sem).wait()

        s_issue(0, dsem0)

        def body(j, carry):
            k0 = 2 * j
            s_issue(k0 + 1, dsem1)
            s_wait(k0, dsem0)
            s_issue(k0 + 2, dsem0)
            s_wait(k0 + 1, dsem1)
            return carry

        lax.fori_loop(0, (N_CHUNKS - 1) // 2, body, 0)
        s_wait(N_CHUNKS - 1, dsem0)
        plsc.subcore_barrier()
        pltpu.sync_copy(
            deg_sh.at[pl.ds(sid * DEG_PER_TILE, DEG_PER_TILE)],
            out_hbm.at[pl.ds(cid * DEG_PAD + sid * DEG_PER_TILE,
                             DEG_PER_TILE)])

    return deg_kernel(edge4).reshape(NC, DEG_PAD)


def _sc_aggregate(y, edge4, f):
    nbuf = 3 if f >= 128 else 6
    tail = N_CHUNKS % nbuf or nbuf
    n_rounds = (N_CHUNKS - tail) // nbuf

    @functools.partial(
        pl.kernel,
        out_type=jax.ShapeDtypeStruct((NC * N, f), jnp.float32),
        mesh=plsc.VectorSubcoreMesh(**_MESH),
        compiler_params=pltpu.CompilerParams(use_tc_tiling_on_sc=False),
        scratch_types=(
            [pltpu.VMEM((N_CHUNKS, EK), jnp.int32)] * 2
            + [pltpu.VMEM((EK, f), jnp.float32)
               for _ in range(nbuf)]
            + [pltpu.VMEM_SHARED((N, f), jnp.float32)]
            + [pltpu.SemaphoreType.DMA] * (2 * nbuf)
        ),
    )
    def agg_kernel(y_hbm, edge_hbm, out_hbm, src_all, dst_all, *bufs):
        rows = list(bufs[:nbuf])
        agg_sh = bufs[nbuf]
        gsem = list(bufs[nbuf + 1:2 * nbuf + 1])
        ssem = list(bufs[2 * nbuf + 1:3 * nbuf + 1])
        cid = lax.axis_index("c")
        sid = lax.axis_index("s")
        w = cid * NS + sid
        r0 = sid * R_BIG

        pltpu.sync_copy(edge_hbm.at[0, w], src_all)
        pltpu.sync_copy(edge_hbm.at[1, w], dst_all)

        def g_issue(k, s):
            pltpu.async_copy(y_hbm.at[src_all.at[k]], rows[s], gsem[s])

        def g_wait(k, s):
            pltpu.make_async_copy(y_hbm.at[src_all.at[k]], rows[s],
                                  gsem[s]).wait()

        def s_issue(k, s):
            pltpu.async_copy(rows[s], agg_sh.at[dst_all.at[k]], ssem[s],
                             add=True)

        def s_wait(k, s):
            pltpu.make_async_copy(rows[s], agg_sh.at[dst_all.at[k]],
                                  ssem[s]).wait()

        for s in range(nbuf):
            g_issue(s, s)

        @pl.when(sid < NS - 1)
        def _():
            pltpu.sync_copy(y_hbm.at[pl.ds(r0, R_BIG)],
                            agg_sh.at[pl.ds(r0, R_BIG)])

        @pl.when(sid == NS - 1)
        def _():
            pltpu.sync_copy(y_hbm.at[pl.ds((NS - 1) * R_BIG, R_LAST)],
                            agg_sh.at[pl.ds((NS - 1) * R_BIG, R_LAST)])

        plsc.subcore_barrier()

        def body(j, carry):
            k0 = nbuf * j
            for s in range(nbuf):
                g_wait(k0 + s, s)
                s_issue(k0 + s, s)
            for s in range(nbuf):
                s_wait(k0 + s, s)
                if s < tail:
                    g_issue(k0 + nbuf + s, s)
                else:
                    @pl.when(j < n_rounds - 1)
                    def _():
                        g_issue(k0 + nbuf + s, s)
            return carry

        lax.fori_loop(0, n_rounds, body, 0)
        for t in range(tail):
            k = N_CHUNKS - tail + t
            g_wait(k, t)
            pltpu.sync_copy(rows[t], agg_sh.at[dst_all.at[k]], add=True)
        plsc.subcore_barrier()

        @pl.when(sid < NS - 1)
        def _():
            pltpu.sync_copy(agg_sh.at[pl.ds(r0, R_BIG)],
                            out_hbm.at[pl.ds(cid * N + r0, R_BIG)])

        @pl.when(sid == NS - 1)
        def _():
            pltpu.sync_copy(
                agg_sh.at[pl.ds((NS - 1) * R_BIG, R_LAST)],
                out_hbm.at[pl.ds(cid * N + (NS - 1) * R_BIG, R_LAST)])

    return agg_kernel(y, edge4)


_BLK = 2000
_GRID = N // _BLK


def _dinv_block(d_ref):
    return lax.rsqrt(d_ref[:, 0:1] + d_ref[:, 1:2] + 1.0)


def _tc_scale(x, deg_col):
    def body(d_ref, x_ref, u_ref):
        u_ref[...] = x_ref[...] * _dinv_block(d_ref)

    return pl.pallas_call(
        body,
        grid=(_GRID,),
        in_specs=[
            pl.BlockSpec((_BLK, 2), lambda i: (i, 0)),
            pl.BlockSpec((_BLK, D), lambda i: (i, 0)),
        ],
        out_specs=pl.BlockSpec((_BLK, D), lambda i: (i, 0)),
        out_shape=jax.ShapeDtypeStruct((N, D), jnp.float32),
    )(deg_col, x)


def _tc_mid(agg2, u, deg_col, w1, b1_row, w2p):
    def body(a0_ref, a1_ref, u_ref, d_ref, w1_ref, b1_ref, w2_ref, out_ref):
        dinv = _dinv_block(d_ref)
        v = a0_ref[...] + a1_ref[...] - u_ref[...]
        t = jnp.dot(v, w1_ref[...],
                    preferred_element_type=jnp.float32) * dinv + b1_ref[...]
        h = jnp.maximum(t, 0.0)
        out_ref[...] = jnp.dot(h, w2_ref[...],
                               preferred_element_type=jnp.float32) * dinv

    f_out = w2p.shape[1]
    flat = agg2
    return pl.pallas_call(
        body,
        grid=(_GRID,),
        in_specs=[
            pl.BlockSpec((_BLK, D), lambda i: (i, 0)),
            pl.BlockSpec((_BLK, D), lambda i: (i + _GRID, 0)),
            pl.BlockSpec((_BLK, D), lambda i: (i, 0)),
            pl.BlockSpec((_BLK, 2), lambda i: (i, 0)),
            pl.BlockSpec((D, D), lambda i: (0, 0)),
            pl.BlockSpec((1, D), lambda i: (0, 0)),
            pl.BlockSpec((D, f_out), lambda i: (0, 0)),
        ],
        out_specs=pl.BlockSpec((_BLK, f_out), lambda i: (i, 0)),
        out_shape=jax.ShapeDtypeStruct((N, f_out), jnp.float32),
    )(flat, flat, u, deg_col, w1, b1_row, w2p)


def _tc_combine_final(agg2, y, deg_col, b_row):
    def body(a0_ref, a1_ref, y_ref, d_ref, b_ref, out_ref):
        out_ref[...] = ((a0_ref[...] + a1_ref[...] - y_ref[...])
                        * _dinv_block(d_ref) + b_ref[...])

    f = y.shape[1]
    flat = agg2
    return pl.pallas_call(
        body,
        grid=(_GRID,),
        in_specs=[
            pl.BlockSpec((_BLK, f), lambda i: (i, 0)),
            pl.BlockSpec((_BLK, f), lambda i: (i + _GRID, 0)),
            pl.BlockSpec((_BLK, f), lambda i: (i, 0)),
            pl.BlockSpec((_BLK, 2), lambda i: (i, 0)),
            pl.BlockSpec((1, f), lambda i: (0, 0)),
        ],
        out_specs=pl.BlockSpec((_BLK, f), lambda i: (i, 0)),
        out_shape=jax.ShapeDtypeStruct((N, f), jnp.float32),
    )(flat, flat, y, deg_col, b_row)


def kernel(x, edge_index, W1, b1, W2, b2):
    edge4 = edge_index.reshape(2, NW, N_CHUNKS, EK)

    deg2 = _sc_degree(edge4)
    deg_col = deg2.T

    u = _tc_scale(x, deg_col)
    agg1 = _sc_aggregate(u, edge4, D)

    y2 = _tc_mid(agg1, u, deg_col, W1, b1.reshape(1, H), W2)
    agg2 = _sc_aggregate(y2, edge4, C)

    return _tc_combine_final(agg2, y2, deg_col, b2.reshape(1, C))
